# Initial kernel scaffold; baseline (speedup 1.0000x reference)
import jax
import jax.numpy as jnp
from jax.experimental import pallas as pl


def kernel(flat_nodes, indexes, conv1_w, conv1_b, conv2_w, conv2_b, conv3_w, conv3_b, dense1_w, dense1_b, dense2_w, dense2_b, out_w, out_b):
    raise NotImplementedError("write your pallas kernel here")


# speedup vs baseline: 936.9569x; 936.9569x over previous
import functools

import jax
import jax.numpy as jnp
from jax.experimental import pallas as pl
from jax.experimental.pallas import tpu as pltpu

BB = 8


def _leaky(x):
    return jnp.where(x >= 0, x, 0.01 * x)


def _kernel(x_ref, idx_ref, w1_ref, b1_ref, w2_ref, b2_ref, w3_ref, b3_ref,
            d1w_ref, d1b_ref, d2w_ref, d2b_ref, ow_ref, ob_ref, o_ref,
            pool_ref):
    n_iota = jax.lax.broadcasted_iota(jnp.int32, (128, 128), 0)
    lane = jax.lax.broadcasted_iota(jnp.int32, (1, 128), 1)
    mask = (lane != 0).astype(jnp.float32)

    for i in range(BB):
        data = x_ref[i]
        idxb = idx_ref[i]
        s_parts = [(n_iota == idxb[k][None, :]).astype(jnp.float32)
                   for k in range(3)]
        sel = jnp.concatenate(s_parts, axis=0)

        for (w_ref, b_ref) in ((w1_ref, b1_ref), (w2_ref, b2_ref),
                               (w3_ref, b3_ref)):
            w = w_ref[...]
            t = jnp.concatenate(
                [jnp.dot(w[k], data, preferred_element_type=jnp.float32)
                 for k in range(3)], axis=1)
            conv = jnp.dot(t, sel, preferred_element_type=jnp.float32)
            conv = (conv + b_ref[...]) * mask
            n_el = conv.shape[0] * conv.shape[1]
            mu = jnp.sum(conv) / n_el
            d = conv - mu
            var = jnp.sum(d * d) / (n_el - 1)
            data = _leaky(d / (jnp.sqrt(var) + 1e-5))

        pool_ref[i] = jnp.max(data, axis=1)

    p = pool_ref[...]
    h = _leaky(jnp.dot(p, d1w_ref[...], preferred_element_type=jnp.float32)
               + d1b_ref[...])
    h = _leaky(jnp.dot(h, d2w_ref[...], preferred_element_type=jnp.float32)
               + d2b_ref[...])
    o_ref[...] = (jnp.sum(h * ow_ref[...], axis=1, keepdims=True)
                  + ob_ref[...])


@jax.jit
def kernel(flat_nodes, indexes, conv1_w, conv1_b, conv2_w, conv2_b, conv3_w,
           conv3_b, dense1_w, dense1_b, dense2_w, dense2_b, out_w, out_b):
    B, C, N = flat_nodes.shape
    L = indexes.shape[1]
    M = L // 3

    idx = indexes.reshape(B, M, 3).transpose(0, 2, 1)
    idx_t = jnp.pad(idx, ((0, 0), (0, 0), (1, 0)))

    w1 = conv1_w.transpose(2, 0, 1)
    w2 = conv2_w.transpose(2, 0, 1)
    w3 = conv3_w.transpose(2, 0, 1)
    b1 = conv1_b.reshape(-1, 1)
    b2 = conv2_b.reshape(-1, 1)
    b3 = conv3_b.reshape(-1, 1)
    d1w = dense1_w.T
    d1b = dense1_b.reshape(1, -1)
    d2w = jnp.zeros((32, 32), jnp.float32).at[:, :28].set(dense2_w.T)
    d2b = jnp.zeros((1, 32), jnp.float32).at[:, :28].set(dense2_b)
    ow = jnp.zeros((1, 32), jnp.float32).at[:, :28].set(out_w)
    ob = out_b.reshape(1, 1)

    full = lambda shape: pl.BlockSpec(shape, lambda i: (0,) * len(shape))
    grid = B // BB
    return pl.pallas_call(
        _kernel,
        grid=(grid,),
        in_specs=[
            pl.BlockSpec((BB, C, N), lambda i: (i, 0, 0)),
            pl.BlockSpec((BB, 3, 128), lambda i: (i, 0, 0)),
            full(w1.shape), full(b1.shape),
            full(w2.shape), full(b2.shape),
            full(w3.shape), full(b3.shape),
            full(d1w.shape), full(d1b.shape),
            full(d2w.shape), full(d2b.shape),
            full(ow.shape), full(ob.shape),
        ],
        out_specs=pl.BlockSpec((BB, 1), lambda i: (i, 0)),
        out_shape=jax.ShapeDtypeStruct((B, 1), jnp.float32),
        scratch_shapes=[pltpu.VMEM((BB, 32), jnp.float32)],
        compiler_params=pltpu.CompilerParams(
            dimension_semantics=("arbitrary",),
        ),
    )(flat_nodes, idx_t, w1, b1, w2, b2, w3, b3, d1w, d1b, d2w, d2b, ow, ob)


# speedup vs baseline: 2617.4011x; 2.7935x over previous
import jax
import jax.numpy as jnp
from jax.experimental import pallas as pl
from jax.experimental.pallas import tpu as pltpu

BB = 8


def _leaky(x):
    return jnp.where(x >= 0, x, 0.01 * x)


def _mm(a, b):
    return jax.lax.dot_general(a, b, (((1,), (0,)), ((), ())),
                               preferred_element_type=jnp.float32)


def _kernel(x_ref, idx_ref, w1_ref, bm1_ref, w2_ref, bm2_ref, w3_ref, bm3_ref,
            d1w_ref, d1b_ref, d2w_ref, d2b_ref, ow_ref, ob_ref, o_ref):
    n_iota = jax.lax.broadcasted_iota(jnp.int32, (128, 128), 0)

    sels = [[(n_iota == idx_ref[i, k][None, :]).astype(jnp.bfloat16)
             for k in range(3)] for i in range(BB)]

    data = x_ref[...]
    for (w_ref, bm_ref) in ((w1_ref, bm1_ref), (w2_ref, bm2_ref),
                            (w3_ref, bm3_ref)):
        w = w_ref[...]
        convs = []
        for i in range(BB):
            di = data[i]
            acc = _mm(_mm(w[0], di), sels[i][0])
            acc += _mm(_mm(w[1], di), sels[i][1])
            acc += _mm(_mm(w[2], di), sels[i][2])
            convs.append(acc)
        conv = jnp.stack(convs) + bm_ref[...][None]
        n_el = conv.shape[1] * conv.shape[2]
        mu = jnp.sum(conv, axis=(1, 2), keepdims=True) / n_el
        d = conv - mu
        var = jnp.sum(d * d, axis=(1, 2), keepdims=True) / (n_el - 1)
        data = _leaky(d / (jnp.sqrt(var) + 1e-5))

    p = jnp.max(data, axis=2)
    h = _leaky(_mm(p, d1w_ref[...]) + d1b_ref[...])
    h = _leaky(_mm(h, d2w_ref[...]) + d2b_ref[...])
    o_ref[...] = (jnp.sum(h * ow_ref[...], axis=1, keepdims=True)
                  + ob_ref[...])


@jax.jit
def kernel(flat_nodes, indexes, conv1_w, conv1_b, conv2_w, conv2_b, conv3_w,
           conv3_b, dense1_w, dense1_b, dense2_w, dense2_b, out_w, out_b):
    B, C, N = flat_nodes.shape
    L = indexes.shape[1]
    M = L // 3

    idx = indexes.reshape(B, M, 3).transpose(0, 2, 1)
    idx_t = jnp.pad(idx, ((0, 0), (0, 0), (1, 0)), constant_values=-1)

    lane_mask = (jnp.arange(N) != 0).astype(jnp.float32)
    w1 = conv1_w.transpose(2, 0, 1)
    w2 = conv2_w.transpose(2, 0, 1)
    w3 = conv3_w.transpose(2, 0, 1)
    bm1 = conv1_b[:, None] * lane_mask[None, :]
    bm2 = conv2_b[:, None] * lane_mask[None, :]
    bm3 = conv3_b[:, None] * lane_mask[None, :]
    d1w = dense1_w.T
    d1b = dense1_b.reshape(1, -1)
    d2w = jnp.zeros((32, 32), jnp.float32).at[:, :28].set(dense2_w.T)
    d2b = jnp.zeros((1, 32), jnp.float32).at[:, :28].set(dense2_b)
    ow = jnp.zeros((1, 32), jnp.float32).at[:, :28].set(out_w)
    ob = out_b.reshape(1, 1)

    full = lambda shape: pl.BlockSpec(shape, lambda i: (0,) * len(shape))
    grid = B // BB
    return pl.pallas_call(
        _kernel,
        grid=(grid,),
        in_specs=[
            pl.BlockSpec((BB, C, N), lambda i: (i, 0, 0)),
            pl.BlockSpec((BB, 3, 128), lambda i: (i, 0, 0)),
            full(w1.shape), full(bm1.shape),
            full(w2.shape), full(bm2.shape),
            full(w3.shape), full(bm3.shape),
            full(d1w.shape), full(d1b.shape),
            full(d2w.shape), full(d2b.shape),
            full(ow.shape), full(ob.shape),
        ],
        out_specs=pl.BlockSpec((BB, 1), lambda i: (i, 0)),
        out_shape=jax.ShapeDtypeStruct((B, 1), jnp.float32),
        compiler_params=pltpu.CompilerParams(
            dimension_semantics=("parallel",),
        ),
    )(flat_nodes, idx_t, w1, bm1, w2, bm2, w3, bm3, d1w, d1b, d2w, d2b, ow, ob)


# speedup vs baseline: 4048.6525x; 1.5468x over previous
import jax
import jax.numpy as jnp
from jax.experimental import pallas as pl
from jax.experimental.pallas import tpu as pltpu

BB = 8


def _leaky(x):
    return jnp.where(x >= 0, x, 0.01 * x)


def _mm(a, b):
    return jax.lax.dot_general(a, b, (((1,), (0,)), ((), ())),
                               preferred_element_type=jnp.float32)


def _kernel(x_ref, idx_ref, w1_ref, bm1_ref, w2_ref, bm2_ref, w3_ref, bm3_ref,
            d1w_ref, d1b_ref, d2w_ref, d2b_ref, ow_ref, ob_ref, o_ref):
    lane = jax.lax.broadcasted_iota(jnp.int32, (1, 128), 1)
    mask = (lane != 0).astype(jnp.float32)

    data = x_ref[...]
    for (w_ref, bm_ref) in ((w1_ref, bm1_ref), (w2_ref, bm2_ref),
                            (w3_ref, bm3_ref)):
        w = w_ref[...]
        o_ch = w.shape[1]
        convs = []
        for i in range(BB):
            di = data[i]
            acc = None
            for k in range(3):
                t = _mm(w[k], di)
                ix = jnp.broadcast_to(idx_ref[i, k][None, :], (o_ch, 128))
                g = jnp.take_along_axis(t, ix, axis=1)
                acc = g if acc is None else acc + g
            convs.append(acc * mask)
        conv = jnp.stack(convs) + bm_ref[...][None]
        n_el = conv.shape[1] * conv.shape[2]
        mu = jnp.sum(conv, axis=(1, 2), keepdims=True) / n_el
        d = conv - mu
        var = jnp.sum(d * d, axis=(1, 2), keepdims=True) / (n_el - 1)
        data = _leaky(d / (jnp.sqrt(var) + 1e-5))

    p = jnp.max(data, axis=2)
    h = _leaky(_mm(p, d1w_ref[...]) + d1b_ref[...])
    h = _leaky(_mm(h, d2w_ref[...]) + d2b_ref[...])
    o_ref[...] = (jnp.sum(h * ow_ref[...], axis=1, keepdims=True)
                  + ob_ref[...])


@jax.jit
def kernel(flat_nodes, indexes, conv1_w, conv1_b, conv2_w, conv2_b, conv3_w,
           conv3_b, dense1_w, dense1_b, dense2_w, dense2_b, out_w, out_b):
    B, C, N = flat_nodes.shape
    L = indexes.shape[1]
    M = L // 3

    idx = indexes.reshape(B, M, 3).transpose(0, 2, 1)
    idx_t = jnp.pad(idx, ((0, 0), (0, 0), (1, 0)))

    lane_mask = (jnp.arange(N) != 0).astype(jnp.float32)
    w1 = conv1_w.transpose(2, 0, 1)
    w2 = conv2_w.transpose(2, 0, 1)
    w3 = conv3_w.transpose(2, 0, 1)
    bm1 = conv1_b[:, None] * lane_mask[None, :]
    bm2 = conv2_b[:, None] * lane_mask[None, :]
    bm3 = conv3_b[:, None] * lane_mask[None, :]
    d1w = dense1_w.T
    d1b = dense1_b.reshape(1, -1)
    d2w = jnp.zeros((32, 32), jnp.float32).at[:, :28].set(dense2_w.T)
    d2b = jnp.zeros((1, 32), jnp.float32).at[:, :28].set(dense2_b)
    ow = jnp.zeros((1, 32), jnp.float32).at[:, :28].set(out_w)
    ob = out_b.reshape(1, 1)

    full = lambda shape: pl.BlockSpec(shape, lambda i: (0,) * len(shape))
    grid = B // BB
    return pl.pallas_call(
        _kernel,
        grid=(grid,),
        in_specs=[
            pl.BlockSpec((BB, C, N), lambda i: (i, 0, 0)),
            pl.BlockSpec((BB, 3, 128), lambda i: (i, 0, 0)),
            full(w1.shape), full(bm1.shape),
            full(w2.shape), full(bm2.shape),
            full(w3.shape), full(bm3.shape),
            full(d1w.shape), full(d1b.shape),
            full(d2w.shape), full(d2b.shape),
            full(ow.shape), full(ob.shape),
        ],
        out_specs=pl.BlockSpec((BB, 1), lambda i: (i, 0)),
        out_shape=jax.ShapeDtypeStruct((B, 1), jnp.float32),
        compiler_params=pltpu.CompilerParams(
            dimension_semantics=("parallel",),
        ),
    )(flat_nodes, idx_t, w1, bm1, w2, bm2, w3, bm3, d1w, d1b, d2w, d2b, ow, ob)


# speedup vs baseline: 10234.7940x; 2.5280x over previous
import jax
import jax.numpy as jnp
from jax.experimental import pallas as pl
from jax.experimental.pallas import tpu as pltpu

BB = 128


def _leaky(x):
    return jnp.where(x >= 0, x, 0.01 * x)


def _mm(a, b):
    return jax.lax.dot_general(a, b, (((1,), (0,)), ((), ())),
                               preferred_element_type=jnp.float32)


def _kernel(x_ref, idx_ref, w1_ref, bm1_ref, w2_ref, bm2_ref, w3_ref, bm3_ref,
            d1w_ref, d1b_ref, d2w_ref, d2b_ref, ow_ref, ob_ref, o_ref):
    lane = jax.lax.broadcasted_iota(jnp.int32, (1, 128), 1)
    mask = (lane != 0).astype(jnp.float32)

    data = x_ref[...]
    for (w_ref, bm_ref) in ((w1_ref, bm1_ref), (w2_ref, bm2_ref),
                            (w3_ref, bm3_ref)):
        w = w_ref[...]
        o_ch = w.shape[1]
        convs = []
        for i in range(BB):
            di = data[i]
            acc = None
            for k in range(3):
                t = _mm(w[k], di)
                ix = jnp.broadcast_to(idx_ref[i, k][None, :], (o_ch, 128))
                g = jnp.take_along_axis(t, ix, axis=1)
                acc = g if acc is None else acc + g
            convs.append(acc * mask)
        conv = jnp.stack(convs) + bm_ref[...][None]
        n_el = conv.shape[1] * conv.shape[2]
        mu = jnp.sum(conv, axis=(1, 2), keepdims=True) / n_el
        d = conv - mu
        var = jnp.sum(d * d, axis=(1, 2), keepdims=True) / (n_el - 1)
        data = _leaky(d / (jnp.sqrt(var) + 1e-5))

    p = jnp.max(data, axis=2)
    h = _leaky(_mm(p, d1w_ref[...]) + d1b_ref[...])
    h = _leaky(_mm(h, d2w_ref[...]) + d2b_ref[...])
    o_ref[...] = (jnp.sum(h * ow_ref[...], axis=1, keepdims=True)
                  + ob_ref[...])


@jax.jit
def kernel(flat_nodes, indexes, conv1_w, conv1_b, conv2_w, conv2_b, conv3_w,
           conv3_b, dense1_w, dense1_b, dense2_w, dense2_b, out_w, out_b):
    B, C, N = flat_nodes.shape
    L = indexes.shape[1]
    M = L // 3

    idx = indexes.reshape(B, M, 3).transpose(0, 2, 1)
    idx_t = jnp.pad(idx, ((0, 0), (0, 0), (1, 0)))

    lane_mask = (jnp.arange(N) != 0).astype(jnp.float32)
    w1 = conv1_w.transpose(2, 0, 1)
    w2 = conv2_w.transpose(2, 0, 1)
    w3 = conv3_w.transpose(2, 0, 1)
    bm1 = conv1_b[:, None] * lane_mask[None, :]
    bm2 = conv2_b[:, None] * lane_mask[None, :]
    bm3 = conv3_b[:, None] * lane_mask[None, :]
    d1w = dense1_w.T
    d1b = dense1_b.reshape(1, -1)
    d2w = jnp.zeros((32, 32), jnp.float32).at[:, :28].set(dense2_w.T)
    d2b = jnp.zeros((1, 32), jnp.float32).at[:, :28].set(dense2_b)
    ow = jnp.zeros((1, 32), jnp.float32).at[:, :28].set(out_w)
    ob = out_b.reshape(1, 1)

    full = lambda shape: pl.BlockSpec(shape, lambda i: (0,) * len(shape))
    grid = B // BB
    return pl.pallas_call(
        _kernel,
        grid=(grid,),
        in_specs=[
            pl.BlockSpec((BB, C, N), lambda i: (i, 0, 0)),
            pl.BlockSpec((BB, 3, 128), lambda i: (i, 0, 0)),
            full(w1.shape), full(bm1.shape),
            full(w2.shape), full(bm2.shape),
            full(w3.shape), full(bm3.shape),
            full(d1w.shape), full(d1b.shape),
            full(d2w.shape), full(d2b.shape),
            full(ow.shape), full(ob.shape),
        ],
        out_specs=pl.BlockSpec((BB, 1), lambda i: (i, 0)),
        out_shape=jax.ShapeDtypeStruct((B, 1), jnp.float32),
        compiler_params=pltpu.CompilerParams(
            dimension_semantics=("parallel",),
        ),
    )(flat_nodes, idx_t, w1, bm1, w2, bm2, w3, bm3, d1w, d1b, d2w, d2b, ow, ob)


# speedup vs baseline: 10752.2170x; 1.0506x over previous
import jax
import jax.numpy as jnp
from jax.experimental import pallas as pl
from jax.experimental.pallas import tpu as pltpu

BB = 128


def _leaky(x):
    return jnp.where(x >= 0, x, 0.01 * x)


def _mm(a, b):
    return jax.lax.dot_general(a, b, (((1,), (0,)), ((), ())),
                               preferred_element_type=jnp.float32)


def _kernel(x_ref, idx_ref, w1_ref, bm1_ref, w2_ref, bm2_ref, w3_ref, bm3_ref,
            d1w_ref, d1b_ref, d2w_ref, d2b_ref, ow_ref, ob_ref, o_ref):
    lane = jax.lax.broadcasted_iota(jnp.int32, (1, 128), 1)
    mask = (lane != 0).astype(jnp.float32)

    data = x_ref[...]
    for (w_ref, bm_ref) in ((w1_ref, bm1_ref), (w2_ref, bm2_ref),
                            (w3_ref, bm3_ref)):
        w = w_ref[...]
        o_ch = w.shape[1]
        convs = []
        for i in range(BB):
            di = data[i]
            acc = None
            for k in range(3):
                t = _mm(w[k], di)
                ix = jnp.broadcast_to(idx_ref[i, k][None, :], (o_ch, 128))
                g = jnp.take_along_axis(t, ix, axis=1)
                acc = g if acc is None else acc + g
            convs.append(acc * mask)
        conv = jnp.stack(convs) + bm_ref[...][None]
        n_el = conv.shape[1] * conv.shape[2]
        s1 = jnp.sum(conv, axis=1)
        s2 = jnp.sum(conv * conv, axis=1)
        t1 = jnp.sum(s1, axis=1, keepdims=True)
        t2 = jnp.sum(s2, axis=1, keepdims=True)
        mu = t1 / n_el
        var = (t2 - mu * t1) / (n_el - 1)
        rr = 1.0 / (jnp.sqrt(var) + 1e-5)
        z = conv * rr[:, :, None] - (mu * rr)[:, :, None]
        data = jnp.maximum(z, 0.01 * z)

    p = jnp.max(data, axis=2)
    h = _leaky(_mm(p, d1w_ref[...]) + d1b_ref[...])
    h = _leaky(_mm(h, d2w_ref[...]) + d2b_ref[...])
    o_ref[...] = (jnp.sum(h * ow_ref[...], axis=1, keepdims=True)
                  + ob_ref[...])


@jax.jit
def kernel(flat_nodes, indexes, conv1_w, conv1_b, conv2_w, conv2_b, conv3_w,
           conv3_b, dense1_w, dense1_b, dense2_w, dense2_b, out_w, out_b):
    B, C, N = flat_nodes.shape
    L = indexes.shape[1]
    M = L // 3

    idx = indexes.reshape(B, M, 3).transpose(0, 2, 1)
    idx_t = jnp.pad(idx, ((0, 0), (0, 0), (1, 0)))

    lane_mask = (jnp.arange(N) != 0).astype(jnp.float32)
    w1 = conv1_w.transpose(2, 0, 1)
    w2 = conv2_w.transpose(2, 0, 1)
    w3 = conv3_w.transpose(2, 0, 1)
    bm1 = conv1_b[:, None] * lane_mask[None, :]
    bm2 = conv2_b[:, None] * lane_mask[None, :]
    bm3 = conv3_b[:, None] * lane_mask[None, :]
    d1w = dense1_w.T
    d1b = dense1_b.reshape(1, -1)
    d2w = jnp.zeros((32, 32), jnp.float32).at[:, :28].set(dense2_w.T)
    d2b = jnp.zeros((1, 32), jnp.float32).at[:, :28].set(dense2_b)
    ow = jnp.zeros((1, 32), jnp.float32).at[:, :28].set(out_w)
    ob = out_b.reshape(1, 1)

    full = lambda shape: pl.BlockSpec(shape, lambda i: (0,) * len(shape))
    grid = B // BB
    return pl.pallas_call(
        _kernel,
        grid=(grid,),
        in_specs=[
            pl.BlockSpec((BB, C, N), lambda i: (i, 0, 0)),
            pl.BlockSpec((BB, 3, 128), lambda i: (i, 0, 0)),
            full(w1.shape), full(bm1.shape),
            full(w2.shape), full(bm2.shape),
            full(w3.shape), full(bm3.shape),
            full(d1w.shape), full(d1b.shape),
            full(d2w.shape), full(d2b.shape),
            full(ow.shape), full(ob.shape),
        ],
        out_specs=pl.BlockSpec((BB, 1), lambda i: (i, 0)),
        out_shape=jax.ShapeDtypeStruct((B, 1), jnp.float32),
        compiler_params=pltpu.CompilerParams(
            dimension_semantics=("parallel",),
        ),
    )(flat_nodes, idx_t, w1, bm1, w2, bm2, w3, bm3, d1w, d1b, d2w, d2b, ow, ob)


# speedup vs baseline: 10754.3233x; 1.0002x over previous
import jax
import jax.numpy as jnp
from jax.experimental import pallas as pl
from jax.experimental.pallas import tpu as pltpu

BB = 128
NG = 1


def _leaky(x):
    return jnp.where(x >= 0, x, 0.01 * x)


def _mm(a, b):
    return jax.lax.dot_general(a, b, (((1,), (0,)), ((), ())),
                               preferred_element_type=jnp.float32)


def _group_net(data, idxg, mask, layers):
    g_sz = data.shape[0]
    for (w_ref, bm_ref) in layers:
        w = w_ref[...]
        o_ch = w.shape[1]
        convs = []
        for i in range(g_sz):
            di = data[i]
            acc = None
            for k in range(3):
                t = _mm(w[k], di)
                ix = jnp.broadcast_to(idxg[i, k][None, :], (o_ch, 128))
                g = jnp.take_along_axis(t, ix, axis=1)
                acc = g if acc is None else acc + g
            convs.append(acc * mask)
        conv = jnp.stack(convs) + bm_ref[...][None]
        n_el = conv.shape[1] * conv.shape[2]
        s1 = jnp.sum(conv, axis=1)
        s2 = jnp.sum(conv * conv, axis=1)
        t1 = jnp.sum(s1, axis=1, keepdims=True)
        t2 = jnp.sum(s2, axis=1, keepdims=True)
        mu = t1 / n_el
        var = (t2 - mu * t1) / (n_el - 1)
        rr = 1.0 / (jnp.sqrt(var) + 1e-5)
        z = conv * rr[:, :, None] - (mu * rr)[:, :, None]
        data = jnp.maximum(z, 0.01 * z)
    return jnp.max(data, axis=2)


def _kernel(x_ref, idx_ref, w1_ref, bm1_ref, w2_ref, bm2_ref, w3_ref, bm3_ref,
            d1w_ref, d1b_ref, d2w_ref, d2b_ref, ow_ref, ob_ref, o_ref):
    lane = jax.lax.broadcasted_iota(jnp.int32, (1, 128), 1)
    mask = (lane != 0).astype(jnp.float32)
    layers = ((w1_ref, bm1_ref), (w2_ref, bm2_ref), (w3_ref, bm3_ref))

    x = x_ref[...]
    idx = idx_ref[...]
    g_sz = BB // NG
    pools = [_group_net(x[g * g_sz:(g + 1) * g_sz],
                        idx[g * g_sz:(g + 1) * g_sz], mask, layers)
             for g in range(NG)]
    p = jnp.concatenate(pools, axis=0)
    h = _leaky(_mm(p, d1w_ref[...]) + d1b_ref[...])
    h = _leaky(_mm(h, d2w_ref[...]) + d2b_ref[...])
    o_ref[...] = (jnp.sum(h * ow_ref[...], axis=1, keepdims=True)
                  + ob_ref[...])


@jax.jit
def kernel(flat_nodes, indexes, conv1_w, conv1_b, conv2_w, conv2_b, conv3_w,
           conv3_b, dense1_w, dense1_b, dense2_w, dense2_b, out_w, out_b):
    B, C, N = flat_nodes.shape
    L = indexes.shape[1]
    M = L // 3

    idx = indexes.reshape(B, M, 3).transpose(0, 2, 1)
    idx_t = jnp.pad(idx, ((0, 0), (0, 0), (1, 0)))

    lane_mask = (jnp.arange(N) != 0).astype(jnp.float32)
    w1 = conv1_w.transpose(2, 0, 1)
    w2 = conv2_w.transpose(2, 0, 1)
    w3 = conv3_w.transpose(2, 0, 1)
    bm1 = conv1_b[:, None] * lane_mask[None, :]
    bm2 = conv2_b[:, None] * lane_mask[None, :]
    bm3 = conv3_b[:, None] * lane_mask[None, :]
    d1w = dense1_w.T
    d1b = dense1_b.reshape(1, -1)
    d2w = jnp.zeros((32, 32), jnp.float32).at[:, :28].set(dense2_w.T)
    d2b = jnp.zeros((1, 32), jnp.float32).at[:, :28].set(dense2_b)
    ow = jnp.zeros((1, 32), jnp.float32).at[:, :28].set(out_w)
    ob = out_b.reshape(1, 1)

    full = lambda shape: pl.BlockSpec(shape, lambda i: (0,) * len(shape))
    grid = B // BB
    return pl.pallas_call(
        _kernel,
        grid=(grid,),
        in_specs=[
            pl.BlockSpec((BB, C, N), lambda i: (i, 0, 0)),
            pl.BlockSpec((BB, 3, 128), lambda i: (i, 0, 0)),
            full(w1.shape), full(bm1.shape),
            full(w2.shape), full(bm2.shape),
            full(w3.shape), full(bm3.shape),
            full(d1w.shape), full(d1b.shape),
            full(d2w.shape), full(d2b.shape),
            full(ow.shape), full(ob.shape),
        ],
        out_specs=pl.BlockSpec((BB, 1), lambda i: (i, 0)),
        out_shape=jax.ShapeDtypeStruct((B, 1), jnp.float32),
        compiler_params=pltpu.CompilerParams(
            dimension_semantics=("parallel",),
        ),
    )(flat_nodes, idx_t, w1, bm1, w2, bm2, w3, bm3, d1w, d1b, d2w, d2b, ow, ob)
